# baseline (speedup 1.0000x reference)
import jax
import jax.numpy as jnp
from jax.experimental import pallas as pl
from jax.experimental.pallas import tpu as pltpu

_N = 1000
_F = 16
_B = 1024
_NB = 128
_G = -(-_N // _NB)


def _broadcast_body(w_ref, out_ref, wt_ref):
    i = pl.program_id(0)

    @pl.when(i == 0)
    def _():
        wt_ref[pl.ds(0, _N), :] = w_ref[...].T

    chunk = wt_ref[pl.ds(i * _NB, _NB), :]
    out_ref[...] = jnp.broadcast_to(chunk[:, :, None], (_NB, _F, _B))


@jax.jit
def kernel(x, embed_weight):
    del x
    out_t = pl.pallas_call(
        _broadcast_body,
        grid=(_G,),
        in_specs=[pl.BlockSpec((_F, _N), lambda i: (0, 0))],
        out_specs=pl.BlockSpec((_NB, _F, _B), lambda i: (i, 0, 0)),
        out_shape=jax.ShapeDtypeStruct((_N, _F, _B), jnp.float32),
        scratch_shapes=[pltpu.VMEM((1024, _F), jnp.float32)],
    )(embed_weight.T)
    return jnp.transpose(out_t, (2, 0, 1))


# speedup vs baseline: 1.0248x; 1.0248x over previous
import jax
import jax.numpy as jnp
from jax.experimental import pallas as pl
from jax.experimental.pallas import tpu as pltpu

_N = 1000
_F = 16
_B = 1024
_NB = 48
_G = -(-_N // _NB)


def _broadcast_body(w_ref, out_ref, wt_ref):
    i = pl.program_id(0)

    @pl.when(i == 0)
    def _():
        wt_ref[pl.ds(0, _N), :] = w_ref[...].T

    chunk = wt_ref[pl.ds(i * _NB, _NB), :]
    out_ref[...] = jnp.broadcast_to(chunk[:, :, None], (_NB, _F, _B))


@jax.jit
def kernel(x, embed_weight):
    del x
    out_t = pl.pallas_call(
        _broadcast_body,
        grid=(_G,),
        in_specs=[pl.BlockSpec((_F, _N), lambda i: (0, 0))],
        out_specs=pl.BlockSpec((_NB, _F, _B), lambda i: (i, 0, 0)),
        out_shape=jax.ShapeDtypeStruct((_N, _F, _B), jnp.float32),
        scratch_shapes=[pltpu.VMEM((1024, _F), jnp.float32)],
    )(embed_weight.T)
    return jnp.transpose(out_t, (2, 0, 1))


# speedup vs baseline: 1.0495x; 1.0241x over previous
import jax
import jax.numpy as jnp
from jax.experimental import pallas as pl
from jax.experimental.pallas import tpu as pltpu

_N = 1000
_F = 16
_B = 1024
_NB = 72
_G = -(-_N // _NB)


def _broadcast_body(w_ref, out_ref, wt_ref):
    i = pl.program_id(0)

    @pl.when(i == 0)
    def _():
        wt_ref[pl.ds(0, _N), :] = w_ref[...].T

    chunk = wt_ref[pl.ds(i * _NB, _NB), :]
    out_ref[...] = jnp.broadcast_to(chunk[:, :, None], (_NB, _F, _B))


@jax.jit
def kernel(x, embed_weight):
    del x
    out_t = pl.pallas_call(
        _broadcast_body,
        grid=(_G,),
        in_specs=[pl.BlockSpec((_F, _N), lambda i: (0, 0))],
        out_specs=pl.BlockSpec((_NB, _F, _B), lambda i: (i, 0, 0)),
        out_shape=jax.ShapeDtypeStruct((_N, _F, _B), jnp.float32),
        scratch_shapes=[pltpu.VMEM((1024, _F), jnp.float32)],
    )(embed_weight.T)
    return jnp.transpose(out_t, (2, 0, 1))


# speedup vs baseline: 1.0538x; 1.0040x over previous
import jax
import jax.numpy as jnp
from jax.experimental import pallas as pl
from jax.experimental.pallas import tpu as pltpu

_N = 1000
_F = 16
_B = 1024
_NB = 64
_G = -(-_N // _NB)


def _broadcast_body(w_ref, out_ref, wt_ref):
    i = pl.program_id(0)

    @pl.when(i == 0)
    def _():
        wt_ref[pl.ds(0, _N), :] = w_ref[...].T

    chunk = wt_ref[pl.ds(i * _NB, _NB), :]
    out_ref[...] = jnp.broadcast_to(chunk[:, :, None], (_NB, _F, _B))


@jax.jit
def kernel(x, embed_weight):
    del x
    out_t = pl.pallas_call(
        _broadcast_body,
        grid=(_G,),
        in_specs=[pl.BlockSpec((_F, _N), lambda i: (0, 0))],
        out_specs=pl.BlockSpec((_NB, _F, _B), lambda i: (i, 0, 0)),
        out_shape=jax.ShapeDtypeStruct((_N, _F, _B), jnp.float32),
        scratch_shapes=[pltpu.VMEM((1024, _F), jnp.float32)],
    )(embed_weight.T)
    return jnp.transpose(out_t, (2, 0, 1))


# speedup vs baseline: 1.0877x; 1.0322x over previous
import jax
import jax.numpy as jnp
from jax.experimental import pallas as pl
from jax.experimental.pallas import tpu as pltpu

_N = 1000
_F = 16
_B = 1024
_NB = 64
_G = -(-_N // _NB)
_LAST = _N - (_G - 1) * _NB


def _broadcast_body(w_ref, out_hbm, buf_a, buf_b, sem_a, sem_b, wt_ref):
    wt_ref[pl.ds(0, _N), :] = w_ref[...].T
    bufs = (buf_a, buf_b)
    sems = (sem_a, sem_b)
    copies = []
    for c in range(_G):
        buf = bufs[c % 2]
        sem = sems[c % 2]
        rows = _NB if c < _G - 1 else _LAST
        if c >= 2:
            copies[c - 2].wait()
        buf[...] = jnp.broadcast_to(
            wt_ref[pl.ds(c * _NB, _NB), :][:, :, None], (_NB, _F, _B)
        )
        d = pltpu.make_async_copy(
            buf.at[pl.ds(0, rows)], out_hbm.at[pl.ds(c * _NB, rows)], sem
        )
        d.start()
        copies.append(d)
    copies[_G - 2].wait()
    copies[_G - 1].wait()


@jax.jit
def kernel(x, embed_weight):
    del x
    out_t = pl.pallas_call(
        _broadcast_body,
        in_specs=[pl.BlockSpec(memory_space=pltpu.VMEM)],
        out_specs=pl.BlockSpec(memory_space=pl.ANY),
        out_shape=jax.ShapeDtypeStruct((_N, _F, _B), jnp.float32),
        scratch_shapes=[
            pltpu.VMEM((_NB, _F, _B), jnp.float32),
            pltpu.VMEM((_NB, _F, _B), jnp.float32),
            pltpu.SemaphoreType.DMA,
            pltpu.SemaphoreType.DMA,
            pltpu.VMEM((1024, _F), jnp.float32),
        ],
    )(embed_weight.T)
    return jnp.transpose(out_t, (2, 0, 1))
